# baseline (speedup 1.0000x reference)
import functools

import jax
import jax.numpy as jnp
from jax import lax
from jax.experimental import pallas as pl
from jax.experimental.pallas import tpu as pltpu
from jax.experimental.pallas import tpu_sc as plsc

_NC = 2
_NS = 16
_NW = _NC * _NS
_C = 1024
_KC = 4


def _convert(tableT, V, D):
    n_cols = V // 128
    Vp = ((V + 127) // 128) * 128
    cw = 128 * _KC
    steps = n_cols // (_KC * _NW)
    xtra = n_cols - steps * _NW * _KC

    mesh = plsc.VectorSubcoreMesh(core_axis_name="c", subcore_axis_name="s")

    @functools.partial(
        pl.kernel,
        mesh=mesh,
        out_type=jax.ShapeDtypeStruct((Vp // 8, 128), jnp.float32),
        scratch_types=[
            pltpu.VMEM((2, D, cw), jnp.float32),
            pltpu.VMEM((2, cw // 8, 128), jnp.float32),
            [pltpu.SemaphoreType.DMA] * 2,
            [pltpu.SemaphoreType.DMA] * 2,
        ],
        compiler_params=pltpu.CompilerParams(
            use_tc_tiling_on_sc=True, needs_layout_passes=False),
    )
    def conv(tab_hbm, lin_hbm, cin, cout, rsems, wsems):
        sid = lax.axis_index("s")
        wid = sid * _NC + lax.axis_index("c")
        iota = lax.iota(jnp.int32, 16)

        def transpose_block(b):
            def body(v0, _):
                for u in range(4):
                    v = v0 * 4 + u
                    fv = jnp.full((16,), v, jnp.int32)
                    row = plsc.load_gather(cin.at[b], [iota, fv])
                    r8 = v // 8
                    c0 = (v % 8) * 16
                    plsc.store_scatter(
                        cout.at[b], [jnp.full((16,), r8, jnp.int32), c0 + iota],
                        row)
                return 0
            lax.fori_loop(0, cw // 4, body, 0)

        def conv_read(v0, b):
            return pltpu.async_copy(
                tab_hbm.at[:, pl.ds(pl.multiple_of(v0, 128), cw)],
                cin.at[b], rsems[b])

        def conv_write(v0, b):
            return pltpu.async_copy(
                cout.at[b],
                lin_hbm.at[pl.ds(pl.multiple_of(v0 // 8, 8), cw // 8)],
                wsems[b])

        v_base = wid * (steps * cw)

        def conv_pair(p, _):
            v0 = v_base + p * (2 * cw)
            r0 = conv_read(v0, 0)
            r1 = conv_read(v0 + cw, 1)
            r0.wait()
            transpose_block(0)
            w0 = conv_write(v0, 0)
            r1.wait()
            transpose_block(1)
            w1 = conv_write(v0 + cw, 1)
            w0.wait()
            w1.wait()
            return 0

        lax.fori_loop(0, steps // 2, conv_pair, 0)
        if steps % 2:
            v0 = v_base + (steps - 1) * cw
            conv_read(v0, 0).wait()
            transpose_block(0)
            conv_write(v0, 0).wait()

        @pl.when(wid == _NW - 1)
        def _tail():
            n_units = xtra + (1 if Vp > n_cols * 128 else 0)
            for j in range(n_units):
                v0 = _NW * steps * cw + j * 128 + 0 * sid
                pltpu.async_copy(tab_hbm.at[:, pl.ds(pl.multiple_of(v0, 128), 128)],
                                 cin.at[0, :, pl.ds(0, 128)], rsems[0]).wait()

                def bodyt(v5, _):
                    for u in range(4):
                        v = v5 * 4 + u
                        fv = jnp.full((16,), v, jnp.int32)
                        row = plsc.load_gather(cin.at[0], [iota, fv])
                        plsc.store_scatter(
                            cout.at[0],
                            [jnp.full((16,), v // 8, jnp.int32),
                             (v % 8) * 16 + iota], row)
                    return 0
                lax.fori_loop(0, 32, bodyt, 0)
                pltpu.async_copy(cout.at[0, pl.ds(0, 16)],
                                 lin_hbm.at[pl.ds(pl.multiple_of(v0 // 8, 8), 16)],
                                 wsems[0]).wait()

    return conv(tableT)


def _gather(idx1, lin, B, Vp, D):
    b_per_w = B // _NW
    n_chunks = b_per_w // _C
    n_t = _C // 128
    n_tc = B // 128

    mesh = plsc.VectorSubcoreMesh(core_axis_name="c", subcore_axis_name="s")

    @functools.partial(
        pl.kernel,
        mesh=mesh,
        out_type=jax.ShapeDtypeStruct((2, n_tc, 8, 128), jnp.float32),
        scratch_types=[
            pltpu.VMEM((2, _C), jnp.int32),
            pltpu.VMEM((2, _C, D), jnp.float32),
            pltpu.VMEM((2, 2, n_t, 8, 128), jnp.float32),
            [pltpu.SemaphoreType.DMA] * 2,
            [pltpu.SemaphoreType.DMA] * 2,
        ],
        compiler_params=pltpu.CompilerParams(
            use_tc_tiling_on_sc=False, needs_layout_passes=False),
    )
    def emb(idx_hbm, lin_hbm, out_hbm, idx_v, rows_v, tile_v, gsems, ssems):
        wid = lax.axis_index("s") * _NC + lax.axis_index("c")
        base = wid * b_per_w
        iota = lax.iota(jnp.int32, 16)
        pre_m = [16 * m + iota for m in range(8)]

        def load_idx(g, b):
            pltpu.sync_copy(idx_hbm.at[pl.ds(base + g * _C, _C)], idx_v.at[b])

        def fire_gather(b):
            return pltpu.async_copy(
                lin_hbm.at[idx_v.at[b]], rows_v.at[b], gsems[b])

        def fire_stores(g, b):
            tc0 = (base + g * _C) // 128
            return [
                pltpu.async_copy(tile_v.at[b, r],
                                 out_hbm.at[r, pl.ds(tc0, n_t)], ssems[b])
                for r in (0, 1)
            ]

        def retile(b):
            rows = rows_v.at[b]

            def body(k, _):
                ridx = [k * 128 + pre_m[m] for m in range(8)]
                for r in (0, 1):
                    for s in range(8):
                        f = jnp.full((16,), 8 * r + s, jnp.int32)
                        for m in range(8):
                            v = plsc.load_gather(rows, [ridx[m], f])
                            tile_v[b, r, k, s, pl.ds(16 * m, 16)] = v
                return 0

            lax.fori_loop(0, n_t, body, 0)

        def pair(p, _):
            g0 = p * 2
            load_idx(g0, 0)
            cg0 = fire_gather(0)
            load_idx(g0 + 1, 1)
            cg1 = fire_gather(1)
            cg0.wait()
            retile(0)
            s0 = fire_stores(g0, 0)
            cg1.wait()
            retile(1)
            s1 = fire_stores(g0 + 1, 1)
            for cp in s0 + s1:
                cp.wait()
            return 0

        lax.fori_loop(0, n_chunks // 2, pair, 0)
        if n_chunks % 2:
            g0 = n_chunks - 1
            load_idx(g0, 0)
            fire_gather(0).wait()
            retile(0)
            for cp in fire_stores(g0, 0):
                cp.wait()

    return emb(idx1, lin)


@jax.jit
def kernel(indices, table):
    (B,) = indices.shape
    V, D = table.shape
    Vp = ((V + 127) // 128) * 128

    idx1 = indices.astype(jnp.int32)
    tableT = table.T

    lin2d = _convert(tableT, V, D)
    lin = lin2d.reshape(Vp, D)
    out4d = _gather(idx1, lin, B, Vp, D)
    return jnp.transpose(out4d, (1, 3, 0, 2)).reshape(B, D)


# speedup vs baseline: 1.2804x; 1.2804x over previous
import functools

import jax
import jax.numpy as jnp
from jax import lax
from jax.experimental import pallas as pl
from jax.experimental.pallas import tpu as pltpu
from jax.experimental.pallas import tpu_sc as plsc

_NC = 2
_NS = 16
_NW = _NC * _NS
_C = 1024


def _convert(tableT, V, D):
    bn = 4096
    G = bn // 8
    grid = (V + bn - 1) // bn
    R = G * grid

    def body(x_ref, o_ref):
        xt = x_ref[...].T
        o_ref[...] = jnp.concatenate(
            [xt[G * k:G * (k + 1), :] for k in range(8)], axis=1)

    return pl.pallas_call(
        body,
        grid=(grid,),
        in_specs=[pl.BlockSpec((D, bn), lambda i: (0, i))],
        out_specs=pl.BlockSpec((G, 128), lambda i: (i, 0)),
        out_shape=jax.ShapeDtypeStruct((R, 128), jnp.float32),
    )(tableT)


def _granule_idx(idx1):
    bi = 102400
    (B,) = idx1.shape

    def body(x_ref, o_ref):
        v = x_ref[...]
        o_ref[...] = (v & ~4095) | ((v & 511) << 3) | ((v >> 9) & 7)

    return pl.pallas_call(
        body,
        grid=(B // bi,),
        in_specs=[pl.BlockSpec((bi,), lambda i: (i,))],
        out_specs=pl.BlockSpec((bi,), lambda i: (i,)),
        out_shape=jax.ShapeDtypeStruct((B,), jnp.int32),
    )(idx1)


def _gather(idx1, lin, B, Vp, D):
    b_per_w = B // _NW
    n_chunks = b_per_w // _C
    n_t = _C // 128
    n_tc = B // 128

    mesh = plsc.VectorSubcoreMesh(core_axis_name="c", subcore_axis_name="s")

    @functools.partial(
        pl.kernel,
        mesh=mesh,
        out_type=jax.ShapeDtypeStruct((2, n_tc, 8, 128), jnp.float32),
        scratch_types=[
            pltpu.VMEM((2, _C), jnp.int32),
            pltpu.VMEM((2, _C, D), jnp.float32),
            pltpu.VMEM((2, 2, n_t, 8, 128), jnp.float32),
            [pltpu.SemaphoreType.DMA] * 2,
            [pltpu.SemaphoreType.DMA] * 2,
        ],
        compiler_params=pltpu.CompilerParams(
            use_tc_tiling_on_sc=False, needs_layout_passes=False),
    )
    def emb(idx_hbm, lin_hbm, out_hbm, idx_v, rows_v, tile_v, gsems, ssems):
        wid = lax.axis_index("s") * _NC + lax.axis_index("c")
        base = wid * b_per_w
        iota = lax.iota(jnp.int32, 16)
        pre_m = [16 * m + iota for m in range(8)]

        def load_idx(g, b):
            pltpu.sync_copy(idx_hbm.at[pl.ds(base + g * _C, _C)], idx_v.at[b])

        def fire_gather(b):
            return pltpu.async_copy(
                lin_hbm.at[idx_v.at[b]], rows_v.at[b], gsems[b])

        def fire_stores(g, b):
            tc0 = (base + g * _C) // 128
            return [
                pltpu.async_copy(tile_v.at[b, r],
                                 out_hbm.at[r, pl.ds(tc0, n_t)], ssems[b])
                for r in (0, 1)
            ]

        def retile(b):
            rows = rows_v.at[b]

            def body(k, _):
                ridx = [k * 128 + pre_m[m] for m in range(8)]
                for r in (0, 1):
                    for s in range(8):
                        f = jnp.full((16,), 8 * r + s, jnp.int32)
                        for m in range(8):
                            v = plsc.load_gather(rows, [ridx[m], f])
                            tile_v[b, r, k, s, pl.ds(16 * m, 16)] = v
                return 0

            lax.fori_loop(0, n_t, body, 0)

        def pair(p, _):
            g0 = p * 2
            load_idx(g0, 0)
            cg0 = fire_gather(0)
            load_idx(g0 + 1, 1)
            cg1 = fire_gather(1)
            cg0.wait()
            retile(0)
            s0 = fire_stores(g0, 0)
            cg1.wait()
            retile(1)
            s1 = fire_stores(g0 + 1, 1)
            for cp in s0 + s1:
                cp.wait()
            return 0

        lax.fori_loop(0, n_chunks // 2, pair, 0)
        if n_chunks % 2:
            g0 = n_chunks - 1
            load_idx(g0, 0)
            fire_gather(0).wait()
            retile(0)
            for cp in fire_stores(g0, 0):
                cp.wait()

    return emb(idx1, lin)


@jax.jit
def kernel(indices, table):
    (B,) = indices.shape
    V, D = table.shape
    Vp = ((V + 127) // 128) * 128

    idx1 = indices.astype(jnp.int32)
    tableT = table.T

    idx2 = _granule_idx(idx1)
    lin2d = _convert(tableT, V, D)
    Vp2 = lin2d.shape[0] * 8
    lin = lin2d.reshape(Vp2, D)
    out4d = _gather(idx2, lin, B, Vp2, D)
    return jnp.transpose(out4d, (1, 3, 0, 2)).reshape(B, D)


# speedup vs baseline: 1.6383x; 1.2796x over previous
import functools

import jax
import jax.numpy as jnp
from jax import lax
from jax.experimental import pallas as pl
from jax.experimental.pallas import tpu as pltpu
from jax.experimental.pallas import tpu_sc as plsc

_NC = 2
_NS = 16
_NW = _NC * _NS
_C = 1024


def _convert(tableT, V, D):
    bn = 4096
    G = bn // 8
    grid = (V + bn - 1) // bn
    R = G * grid

    def body(x_ref, o_ref):
        x = x_ref[...]
        xx = jnp.concatenate(
            [x[:, G * k:G * (k + 1)] for k in range(8)], axis=0)
        o_ref[...] = xx.T

    return pl.pallas_call(
        body,
        grid=(grid,),
        in_specs=[pl.BlockSpec((D, bn), lambda i: (0, i))],
        out_specs=pl.BlockSpec((G, 128), lambda i: (i, 0)),
        out_shape=jax.ShapeDtypeStruct((R, 128), jnp.float32),
    )(tableT)


def _granule_idx(idx1):
    bi = 102400
    (B,) = idx1.shape

    def body(x_ref, o_ref):
        v = x_ref[...]
        o_ref[...] = (v & ~4095) | ((v & 511) << 3) | ((v >> 9) & 7)

    return pl.pallas_call(
        body,
        grid=(B // bi,),
        in_specs=[pl.BlockSpec((bi,), lambda i: (i,))],
        out_specs=pl.BlockSpec((bi,), lambda i: (i,)),
        out_shape=jax.ShapeDtypeStruct((B,), jnp.int32),
    )(idx1)


def _gather(idx1, lin, B, Vp, D):
    b_per_w = B // _NW
    n_chunks = b_per_w // _C
    n_t = _C // 128
    n_tc = B // 128

    mesh = plsc.VectorSubcoreMesh(core_axis_name="c", subcore_axis_name="s")

    @functools.partial(
        pl.kernel,
        mesh=mesh,
        out_type=jax.ShapeDtypeStruct((2, n_tc, 8, 128), jnp.float32),
        scratch_types=[
            pltpu.VMEM((2, _C), jnp.int32),
            pltpu.VMEM((2, _C, D), jnp.float32),
            pltpu.VMEM((2, 2, n_t, 8, 128), jnp.float32),
            [pltpu.SemaphoreType.DMA] * 2,
            [pltpu.SemaphoreType.DMA] * 2,
        ],
        compiler_params=pltpu.CompilerParams(
            use_tc_tiling_on_sc=False, needs_layout_passes=False),
    )
    def emb(idx_hbm, lin_hbm, out_hbm, idx_v, rows_v, tile_v, gsems, ssems):
        wid = lax.axis_index("s") * _NC + lax.axis_index("c")
        base = wid * b_per_w
        iota = lax.iota(jnp.int32, 16)
        pre_m = [16 * m + iota for m in range(8)]

        def load_idx(g, b):
            pltpu.sync_copy(idx_hbm.at[pl.ds(base + g * _C, _C)], idx_v.at[b])

        def fire_gather(b):
            return pltpu.async_copy(
                lin_hbm.at[idx_v.at[b]], rows_v.at[b], gsems[b])

        def fire_stores(g, b):
            tc0 = (base + g * _C) // 128
            return [
                pltpu.async_copy(tile_v.at[b, r],
                                 out_hbm.at[r, pl.ds(tc0, n_t)], ssems[b])
                for r in (0, 1)
            ]

        def retile(b):
            rows = rows_v.at[b]

            def body(k, _):
                ridx = [k * 128 + pre_m[m] for m in range(8)]
                for r in (0, 1):
                    for s in range(8):
                        f = jnp.full((16,), 8 * r + s, jnp.int32)
                        for m in range(8):
                            v = plsc.load_gather(rows, [ridx[m], f])
                            tile_v[b, r, k, s, pl.ds(16 * m, 16)] = v
                return 0

            lax.fori_loop(0, n_t, body, 0)

        def pair(p, _):
            g0 = p * 2
            load_idx(g0, 0)
            cg0 = fire_gather(0)
            load_idx(g0 + 1, 1)
            cg1 = fire_gather(1)
            cg0.wait()
            retile(0)
            s0 = fire_stores(g0, 0)
            cg1.wait()
            retile(1)
            s1 = fire_stores(g0 + 1, 1)
            for cp in s0 + s1:
                cp.wait()
            return 0

        lax.fori_loop(0, n_chunks // 2, pair, 0)
        if n_chunks % 2:
            g0 = n_chunks - 1
            load_idx(g0, 0)
            fire_gather(0).wait()
            retile(0)
            for cp in fire_stores(g0, 0):
                cp.wait()

    return emb(idx1, lin)


@jax.jit
def kernel(indices, table):
    (B,) = indices.shape
    V, D = table.shape
    Vp = ((V + 127) // 128) * 128

    idx1 = indices.astype(jnp.int32)
    tableT = table.T

    idx2 = _granule_idx(idx1)
    lin2d = _convert(tableT, V, D)
    Vp2 = lin2d.shape[0] * 8
    lin = lin2d.reshape(Vp2, D)
    out4d = _gather(idx2, lin, B, Vp2, D)
    return jnp.transpose(out4d, (1, 3, 0, 2)).reshape(B, D)


# speedup vs baseline: 2.1098x; 1.2878x over previous
import functools

import jax
import jax.numpy as jnp
from jax import lax
from jax.experimental import pallas as pl
from jax.experimental.pallas import tpu as pltpu
from jax.experimental.pallas import tpu_sc as plsc

_NC = 2
_NS = 16
_NW = _NC * _NS
_C = 1024


def _convert(tableT, V, D):
    bn = 4096
    G = bn // 8
    grid = (V + bn - 1) // bn
    R = G * grid

    def body(x_ref, o_ref):
        x = x_ref[...]
        xx = jnp.concatenate(
            [x[:, G * k:G * (k + 1)] for k in range(8)], axis=0)
        o_ref[...] = xx.T

    return pl.pallas_call(
        body,
        grid=(grid,),
        in_specs=[pl.BlockSpec((D, bn), lambda i: (0, i))],
        out_specs=pl.BlockSpec((G, 128), lambda i: (i, 0)),
        out_shape=jax.ShapeDtypeStruct((R, 128), jnp.float32),
    )(tableT)


def _granule_idx(idx1):
    bi = 102400
    (B,) = idx1.shape

    def body(x_ref, o_ref):
        v = x_ref[...]
        o_ref[...] = (v & ~4095) | ((v & 511) << 3) | ((v >> 9) & 7)

    return pl.pallas_call(
        body,
        grid=(B // bi,),
        in_specs=[pl.BlockSpec((bi,), lambda i: (i,))],
        out_specs=pl.BlockSpec((bi,), lambda i: (i,)),
        out_shape=jax.ShapeDtypeStruct((B,), jnp.int32),
    )(idx1)


def _gather(idx1, lin, B, Vp, D):
    b_per_w = B // _NW
    n_chunks = b_per_w // _C
    n_t = _C // 128
    n_tc = B // 128

    mesh = plsc.VectorSubcoreMesh(core_axis_name="c", subcore_axis_name="s")

    @functools.partial(
        pl.kernel,
        mesh=mesh,
        out_type=jax.ShapeDtypeStruct((2, n_tc * 8, 128), jnp.float32),
        scratch_types=[
            pltpu.VMEM((2, _C), jnp.int32),
            pltpu.VMEM((2, _C, D), jnp.float32),
            pltpu.VMEM((2, 2 * n_t * 8, 129), jnp.float32),
            [pltpu.SemaphoreType.DMA] * 2,
            [pltpu.SemaphoreType.DMA] * 2,
        ],
        compiler_params=pltpu.CompilerParams(
            use_tc_tiling_on_sc=False, needs_layout_passes=False),
    )
    def emb(idx_hbm, lin_hbm, out_hbm, idx_v, rows_v, tile_v, gsems, ssems):
        wid = lax.axis_index("s") * _NC + lax.axis_index("c")
        base = wid * b_per_w
        iota = lax.iota(jnp.int32, 16)
        row_f = (iota // 8) * (n_t * 8) + (iota % 8)

        def load_idx(g, b):
            pltpu.sync_copy(idx_hbm.at[pl.ds(base + g * _C, _C)], idx_v.at[b])

        def fire_gather(b):
            return pltpu.async_copy(
                lin_hbm.at[idx_v.at[b]], rows_v.at[b], gsems[b])

        def fire_stores(g, b):
            tc0 = (base + g * _C) // 128
            return [
                pltpu.async_copy(
                    tile_v.at[b, pl.ds(r * (n_t * 8), n_t * 8), pl.ds(0, 128)],
                    out_hbm.at[r, pl.ds(tc0 * 8, n_t * 8)], ssems[b])
                for r in (0, 1)
            ]

        def retile(b):
            for k in range(n_t):
                rowv = row_f + k * 8

                def wbody(w4, colv, k=k, rowv=rowv):
                    for u in range(4):
                        v = rows_v[b, k * 128 + w4 * 4 + u, :]
                        plsc.store_scatter(tile_v.at[b], [rowv, colv + u], v)
                    return colv + 4

                lax.fori_loop(0, 32, wbody, jnp.zeros((16,), jnp.int32))

        def pair(p, _):
            g0 = p * 2
            load_idx(g0, 0)
            cg0 = fire_gather(0)
            load_idx(g0 + 1, 1)
            cg1 = fire_gather(1)
            cg0.wait()
            retile(0)
            s0 = fire_stores(g0, 0)
            cg1.wait()
            retile(1)
            s1 = fire_stores(g0 + 1, 1)
            for cp in s0 + s1:
                cp.wait()
            return 0

        lax.fori_loop(0, n_chunks // 2, pair, 0)
        if n_chunks % 2:
            g0 = n_chunks - 1
            load_idx(g0, 0)
            fire_gather(0).wait()
            retile(0)
            for cp in fire_stores(g0, 0):
                cp.wait()

    return emb(idx1, lin)


@jax.jit
def kernel(indices, table):
    (B,) = indices.shape
    V, D = table.shape
    Vp = ((V + 127) // 128) * 128

    idx1 = indices.astype(jnp.int32)
    tableT = table.T

    idx2 = _granule_idx(idx1)
    lin2d = _convert(tableT, V, D)
    Vp2 = lin2d.shape[0] * 8
    lin = lin2d.reshape(Vp2, D)
    out3 = _gather(idx2, lin, B, Vp2, D)
    out4d = out3.reshape(2, B // 128, 8, 128)
    return jnp.transpose(out4d, (1, 3, 0, 2)).reshape(B, D)


# speedup vs baseline: 2.6011x; 1.2329x over previous
import functools

import jax
import jax.numpy as jnp
from jax import lax
from jax.experimental import pallas as pl
from jax.experimental.pallas import tpu as pltpu
from jax.experimental.pallas import tpu_sc as plsc

_NC = 2
_NS = 16
_NW = _NC * _NS
_C = 1024
_BN = 8192


def _convert(tableT, V, D):
    bn = _BN
    G = bn // 8
    grid = (V + bn - 1) // bn
    R = G * grid

    def body(x_ref, o_ref):
        x = x_ref[...]
        xx = jnp.concatenate(
            [x[:, G * k:G * (k + 1)] for k in range(8)], axis=0)
        o_ref[...] = xx.T

    return pl.pallas_call(
        body,
        grid=(grid,),
        in_specs=[pl.BlockSpec((D, bn), lambda i: (0, i))],
        out_specs=pl.BlockSpec((G, 128), lambda i: (i, 0)),
        out_shape=jax.ShapeDtypeStruct((R, 128), jnp.float32),
    )(tableT)


def _granule_idx(idx1):
    bi = 102400
    (B,) = idx1.shape

    G = _BN // 8
    sh = G.bit_length() - 1

    def body(x_ref, o_ref):
        v = x_ref[...]
        o_ref[...] = (v & ~(_BN - 1)) | ((v & (G - 1)) << 3) | ((v >> sh) & 7)

    return pl.pallas_call(
        body,
        grid=(B // bi,),
        in_specs=[pl.BlockSpec((bi,), lambda i: (i,))],
        out_specs=pl.BlockSpec((bi,), lambda i: (i,)),
        out_shape=jax.ShapeDtypeStruct((B,), jnp.int32),
    )(idx1)


def _gather(idx1, lin, B, Vp, D):
    b_per_w = B // _NW
    n_chunks = b_per_w // _C
    n_t = _C // 128
    n_tc = B // 128

    mesh = plsc.VectorSubcoreMesh(core_axis_name="c", subcore_axis_name="s")

    @functools.partial(
        pl.kernel,
        mesh=mesh,
        out_type=jax.ShapeDtypeStruct((2, n_tc * 8, 128), jnp.float32),
        scratch_types=[
            pltpu.VMEM((2, _C), jnp.int32),
            pltpu.VMEM((2, _C, D), jnp.float32),
            pltpu.VMEM((2, 2 * n_t * 8, 129), jnp.float32),
            [pltpu.SemaphoreType.DMA] * 2,
            [pltpu.SemaphoreType.DMA] * 2,
        ],
        compiler_params=pltpu.CompilerParams(
            use_tc_tiling_on_sc=False, needs_layout_passes=False),
    )
    def emb(idx_hbm, lin_hbm, out_hbm, idx_v, rows_v, tile_v, gsems, ssems):
        wid = lax.axis_index("s") * _NC + lax.axis_index("c")
        base = wid * b_per_w
        iota = lax.iota(jnp.int32, 16)
        row_f = (iota // 8) * (n_t * 8) + (iota % 8)

        def load_idx(g, b):
            pltpu.sync_copy(idx_hbm.at[pl.ds(base + g * _C, _C)], idx_v.at[b])

        def fire_gather(b):
            return pltpu.async_copy(
                lin_hbm.at[idx_v.at[b]], rows_v.at[b], gsems[b])

        def fire_stores(g, b):
            tc0 = (base + g * _C) // 128
            return [
                pltpu.async_copy(
                    tile_v.at[b, pl.ds(r * (n_t * 8), n_t * 8), pl.ds(0, 128)],
                    out_hbm.at[r, pl.ds(tc0 * 8, n_t * 8)], ssems[b])
                for r in (0, 1)
            ]

        def retile(b):
            for k in range(n_t):
                rowv = row_f + k * 8

                def wbody(w4, colv, k=k, rowv=rowv):
                    for u in range(4):
                        v = rows_v[b, k * 128 + w4 * 4 + u, :]
                        plsc.store_scatter(tile_v.at[b], [rowv, colv + u], v)
                    return colv + 4

                lax.fori_loop(0, 32, wbody, jnp.zeros((16,), jnp.int32))

        def pair(p, _):
            g0 = p * 2
            load_idx(g0, 0)
            cg0 = fire_gather(0)
            load_idx(g0 + 1, 1)
            cg1 = fire_gather(1)
            cg0.wait()
            retile(0)
            s0 = fire_stores(g0, 0)
            cg1.wait()
            retile(1)
            s1 = fire_stores(g0 + 1, 1)
            for cp in s0 + s1:
                cp.wait()
            return 0

        lax.fori_loop(0, n_chunks // 2, pair, 0)
        if n_chunks % 2:
            g0 = n_chunks - 1
            load_idx(g0, 0)
            fire_gather(0).wait()
            retile(0)
            for cp in fire_stores(g0, 0):
                cp.wait()

    return emb(idx1, lin)


@jax.jit
def kernel(indices, table):
    (B,) = indices.shape
    V, D = table.shape
    Vp = ((V + 127) // 128) * 128

    idx1 = indices.astype(jnp.int32)
    tableT = table.T

    idx2 = _granule_idx(idx1)
    lin2d = _convert(tableT, V, D)
    Vp2 = lin2d.shape[0] * 8
    lin = lin2d.reshape(Vp2, D)
    out3 = _gather(idx2, lin, B, Vp2, D)
    out4d = out3.reshape(2, B // 128, 8, 128)
    return jnp.transpose(out4d, (1, 3, 0, 2)).reshape(B, D)


# speedup vs baseline: 2.9047x; 1.1167x over previous
import functools

import jax
import jax.numpy as jnp
from jax import lax
from jax.experimental import pallas as pl
from jax.experimental.pallas import tpu as pltpu
from jax.experimental.pallas import tpu_sc as plsc

_NC = 2
_NS = 16
_NW = _NC * _NS
_C = 1024
_BN = 16384


def _convert(tableT, V, D):
    bn = _BN
    G = bn // 8
    grid = (V + bn - 1) // bn
    R = G * grid

    def body(x_ref, o_ref):
        x = x_ref[...]
        xx = jnp.concatenate(
            [x[:, G * k:G * (k + 1)] for k in range(8)], axis=0)
        o_ref[...] = xx.T

    return pl.pallas_call(
        body,
        grid=(grid,),
        in_specs=[pl.BlockSpec((D, bn), lambda i: (0, i))],
        out_specs=pl.BlockSpec((G, 128), lambda i: (i, 0)),
        out_shape=jax.ShapeDtypeStruct((R, 128), jnp.float32),
    )(tableT)


def _granule_idx(idx1):
    bi = 102400
    (B,) = idx1.shape

    G = _BN // 8
    sh = G.bit_length() - 1

    def body(x_ref, o_ref):
        v = x_ref[...]
        o_ref[...] = (v & ~(_BN - 1)) | ((v & (G - 1)) << 3) | ((v >> sh) & 7)

    return pl.pallas_call(
        body,
        grid=(B // bi,),
        in_specs=[pl.BlockSpec((bi,), lambda i: (i,))],
        out_specs=pl.BlockSpec((bi,), lambda i: (i,)),
        out_shape=jax.ShapeDtypeStruct((B,), jnp.int32),
    )(idx1)


def _gather(idx1, lin, B, Vp, D):
    b_per_w = B // _NW
    n_chunks = b_per_w // _C
    n_t = _C // 128
    n_tc = B // 128

    mesh = plsc.VectorSubcoreMesh(core_axis_name="c", subcore_axis_name="s")

    @functools.partial(
        pl.kernel,
        mesh=mesh,
        out_type=jax.ShapeDtypeStruct((2, n_tc * 8, 128), jnp.float32),
        scratch_types=[
            pltpu.VMEM((2, _C), jnp.int32),
            pltpu.VMEM((2, _C, D), jnp.float32),
            pltpu.VMEM((2, 2 * n_t * 8, 129), jnp.float32),
            [pltpu.SemaphoreType.DMA] * 2,
            [pltpu.SemaphoreType.DMA] * 2,
        ],
        compiler_params=pltpu.CompilerParams(
            use_tc_tiling_on_sc=False, needs_layout_passes=False),
    )
    def emb(idx_hbm, lin_hbm, out_hbm, idx_v, rows_v, tile_v, gsems, ssems):
        wid = lax.axis_index("s") * _NC + lax.axis_index("c")
        base = wid * b_per_w
        iota = lax.iota(jnp.int32, 16)
        row_f = (iota // 8) * (n_t * 8) + (iota % 8)

        def load_idx(g, b):
            pltpu.sync_copy(idx_hbm.at[pl.ds(base + g * _C, _C)], idx_v.at[b])

        def fire_gather(b):
            return pltpu.async_copy(
                lin_hbm.at[idx_v.at[b]], rows_v.at[b], gsems[b])

        def fire_stores(g, b):
            tc0 = (base + g * _C) // 128
            return [
                pltpu.async_copy(
                    tile_v.at[b, pl.ds(r * (n_t * 8), n_t * 8), pl.ds(0, 128)],
                    out_hbm.at[r, pl.ds(tc0 * 8, n_t * 8)], ssems[b])
                for r in (0, 1)
            ]

        def retile(b):
            for k in range(n_t):
                rowv = row_f + k * 8

                def wbody(w4, colv, k=k, rowv=rowv):
                    for u in range(4):
                        v = rows_v[b, k * 128 + w4 * 4 + u, :]
                        plsc.store_scatter(tile_v.at[b], [rowv, colv + u], v)
                    return colv + 4

                lax.fori_loop(0, 32, wbody, jnp.zeros((16,), jnp.int32))

        def pair(p, _):
            g0 = p * 2
            load_idx(g0, 0)
            cg0 = fire_gather(0)
            load_idx(g0 + 1, 1)
            cg1 = fire_gather(1)
            cg0.wait()
            retile(0)
            s0 = fire_stores(g0, 0)
            cg1.wait()
            retile(1)
            s1 = fire_stores(g0 + 1, 1)
            for cp in s0 + s1:
                cp.wait()
            return 0

        lax.fori_loop(0, n_chunks // 2, pair, 0)
        if n_chunks % 2:
            g0 = n_chunks - 1
            load_idx(g0, 0)
            fire_gather(0).wait()
            retile(0)
            for cp in fire_stores(g0, 0):
                cp.wait()

    return emb(idx1, lin)


@jax.jit
def kernel(indices, table):
    (B,) = indices.shape
    V, D = table.shape
    Vp = ((V + 127) // 128) * 128

    idx1 = indices.astype(jnp.int32)
    tableT = table.T

    idx2 = _granule_idx(idx1)
    lin2d = _convert(tableT, V, D)
    Vp2 = lin2d.shape[0] * 8
    lin = lin2d.reshape(Vp2, D)
    out3 = _gather(idx2, lin, B, Vp2, D)
    out4d = out3.reshape(2, B // 128, 8, 128)
    return jnp.transpose(out4d, (1, 3, 0, 2)).reshape(B, D)


# speedup vs baseline: 3.1487x; 1.0840x over previous
import functools

import jax
import jax.numpy as jnp
from jax import lax
from jax.experimental import pallas as pl
from jax.experimental.pallas import tpu as pltpu
from jax.experimental.pallas import tpu_sc as plsc

_NC = 2
_NS = 16
_NW = _NC * _NS
_C = 1024
_BN = 32768


def _convert(tableT, V, D):
    bn = _BN
    G = bn // 8
    grid = (V + bn - 1) // bn
    R = G * grid

    def body(x_ref, o_ref):
        x = x_ref[...]
        xx = jnp.concatenate(
            [x[:, G * k:G * (k + 1)] for k in range(8)], axis=0)
        o_ref[...] = xx.T

    return pl.pallas_call(
        body,
        grid=(grid,),
        in_specs=[pl.BlockSpec((D, bn), lambda i: (0, i))],
        out_specs=pl.BlockSpec((G, 128), lambda i: (i, 0)),
        out_shape=jax.ShapeDtypeStruct((R, 128), jnp.float32),
    )(tableT)


def _granule_idx(idx1):
    bi = 102400
    (B,) = idx1.shape

    G = _BN // 8
    sh = G.bit_length() - 1

    def body(x_ref, o_ref):
        v = x_ref[...]
        o_ref[...] = (v & ~(_BN - 1)) | ((v & (G - 1)) << 3) | ((v >> sh) & 7)

    return pl.pallas_call(
        body,
        grid=(B // bi,),
        in_specs=[pl.BlockSpec((bi,), lambda i: (i,))],
        out_specs=pl.BlockSpec((bi,), lambda i: (i,)),
        out_shape=jax.ShapeDtypeStruct((B,), jnp.int32),
    )(idx1)


def _gather(idx1, lin, B, Vp, D):
    b_per_w = B // _NW
    n_chunks = b_per_w // _C
    n_t = _C // 128
    n_tc = B // 128

    mesh = plsc.VectorSubcoreMesh(core_axis_name="c", subcore_axis_name="s")

    @functools.partial(
        pl.kernel,
        mesh=mesh,
        out_type=jax.ShapeDtypeStruct((2, n_tc * 8, 128), jnp.float32),
        scratch_types=[
            pltpu.VMEM((2, _C), jnp.int32),
            pltpu.VMEM((2, _C, D), jnp.float32),
            pltpu.VMEM((2, 2 * n_t * 8, 129), jnp.float32),
            [pltpu.SemaphoreType.DMA] * 2,
            [pltpu.SemaphoreType.DMA] * 2,
        ],
        compiler_params=pltpu.CompilerParams(
            use_tc_tiling_on_sc=False, needs_layout_passes=False),
    )
    def emb(idx_hbm, lin_hbm, out_hbm, idx_v, rows_v, tile_v, gsems, ssems):
        wid = lax.axis_index("s") * _NC + lax.axis_index("c")
        base = wid * b_per_w
        iota = lax.iota(jnp.int32, 16)
        row_f = (iota // 8) * (n_t * 8) + (iota % 8)

        def load_idx(g, b):
            pltpu.sync_copy(idx_hbm.at[pl.ds(base + g * _C, _C)], idx_v.at[b])

        def fire_gather(b):
            return pltpu.async_copy(
                lin_hbm.at[idx_v.at[b]], rows_v.at[b], gsems[b])

        def fire_stores(g, b):
            tc0 = (base + g * _C) // 128
            return [
                pltpu.async_copy(
                    tile_v.at[b, pl.ds(r * (n_t * 8), n_t * 8), pl.ds(0, 128)],
                    out_hbm.at[r, pl.ds(tc0 * 8, n_t * 8)], ssems[b])
                for r in (0, 1)
            ]

        def retile(b):
            for k in range(n_t):
                rowv = row_f + k * 8

                def wbody(w4, colv, k=k, rowv=rowv):
                    for u in range(4):
                        v = rows_v[b, k * 128 + w4 * 4 + u, :]
                        plsc.store_scatter(tile_v.at[b], [rowv, colv + u], v)
                    return colv + 4

                lax.fori_loop(0, 32, wbody, jnp.zeros((16,), jnp.int32))

        def pair(p, _):
            g0 = p * 2
            load_idx(g0, 0)
            cg0 = fire_gather(0)
            load_idx(g0 + 1, 1)
            cg1 = fire_gather(1)
            cg0.wait()
            retile(0)
            s0 = fire_stores(g0, 0)
            cg1.wait()
            retile(1)
            s1 = fire_stores(g0 + 1, 1)
            for cp in s0 + s1:
                cp.wait()
            return 0

        lax.fori_loop(0, n_chunks // 2, pair, 0)
        if n_chunks % 2:
            g0 = n_chunks - 1
            load_idx(g0, 0)
            fire_gather(0).wait()
            retile(0)
            for cp in fire_stores(g0, 0):
                cp.wait()

    return emb(idx1, lin)


@jax.jit
def kernel(indices, table):
    (B,) = indices.shape
    V, D = table.shape
    Vp = ((V + 127) // 128) * 128

    idx1 = indices.astype(jnp.int32)
    tableT = table.T

    idx2 = _granule_idx(idx1)
    lin2d = _convert(tableT, V, D)
    Vp2 = lin2d.shape[0] * 8
    lin = lin2d.reshape(Vp2, D)
    out3 = _gather(idx2, lin, B, Vp2, D)
    out4d = out3.reshape(2, B // 128, 8, 128)
    return jnp.transpose(out4d, (1, 3, 0, 2)).reshape(B, D)


# speedup vs baseline: 3.2490x; 1.0318x over previous
import functools

import jax
import jax.numpy as jnp
from jax import lax
from jax.experimental import pallas as pl
from jax.experimental.pallas import tpu as pltpu
from jax.experimental.pallas import tpu_sc as plsc

_NC = 2
_NS = 16
_NW = _NC * _NS
_C = 1024
_BN = 65536


def _convert(tableT, V, D):
    bn = _BN
    G = bn // 8
    grid = (V + bn - 1) // bn
    R = G * grid

    def body(x_ref, o_ref):
        x = x_ref[...]
        xx = jnp.concatenate(
            [x[:, G * k:G * (k + 1)] for k in range(8)], axis=0)
        o_ref[...] = xx.T

    return pl.pallas_call(
        body,
        grid=(grid,),
        in_specs=[pl.BlockSpec((D, bn), lambda i: (0, i))],
        out_specs=pl.BlockSpec((G, 128), lambda i: (i, 0)),
        out_shape=jax.ShapeDtypeStruct((R, 128), jnp.float32),
    )(tableT)


def _granule_idx(idx1):
    bi = 102400
    (B,) = idx1.shape

    G = _BN // 8
    sh = G.bit_length() - 1

    def body(x_ref, o_ref):
        v = x_ref[...]
        o_ref[...] = (v & ~(_BN - 1)) | ((v & (G - 1)) << 3) | ((v >> sh) & 7)

    return pl.pallas_call(
        body,
        grid=(B // bi,),
        in_specs=[pl.BlockSpec((bi,), lambda i: (i,))],
        out_specs=pl.BlockSpec((bi,), lambda i: (i,)),
        out_shape=jax.ShapeDtypeStruct((B,), jnp.int32),
    )(idx1)


def _gather(idx1, lin, B, Vp, D):
    b_per_w = B // _NW
    n_chunks = b_per_w // _C
    n_t = _C // 128
    n_tc = B // 128

    mesh = plsc.VectorSubcoreMesh(core_axis_name="c", subcore_axis_name="s")

    @functools.partial(
        pl.kernel,
        mesh=mesh,
        out_type=jax.ShapeDtypeStruct((2, n_tc * 8, 128), jnp.float32),
        scratch_types=[
            pltpu.VMEM((2, _C), jnp.int32),
            pltpu.VMEM((2, _C, D), jnp.float32),
            pltpu.VMEM((2, 2 * n_t * 8, 129), jnp.float32),
            [pltpu.SemaphoreType.DMA] * 2,
            [pltpu.SemaphoreType.DMA] * 2,
        ],
        compiler_params=pltpu.CompilerParams(
            use_tc_tiling_on_sc=False, needs_layout_passes=False),
    )
    def emb(idx_hbm, lin_hbm, out_hbm, idx_v, rows_v, tile_v, gsems, ssems):
        wid = lax.axis_index("s") * _NC + lax.axis_index("c")
        base = wid * b_per_w
        iota = lax.iota(jnp.int32, 16)
        row_f = (iota // 8) * (n_t * 8) + (iota % 8)

        def load_idx(g, b):
            pltpu.sync_copy(idx_hbm.at[pl.ds(base + g * _C, _C)], idx_v.at[b])

        def fire_gather(b):
            return pltpu.async_copy(
                lin_hbm.at[idx_v.at[b]], rows_v.at[b], gsems[b])

        def fire_stores(g, b):
            tc0 = (base + g * _C) // 128
            return [
                pltpu.async_copy(
                    tile_v.at[b, pl.ds(r * (n_t * 8), n_t * 8), pl.ds(0, 128)],
                    out_hbm.at[r, pl.ds(tc0 * 8, n_t * 8)], ssems[b])
                for r in (0, 1)
            ]

        def retile(b):
            for k in range(n_t):
                rowv = row_f + k * 8

                def wbody(w4, colv, k=k, rowv=rowv):
                    for u in range(4):
                        v = rows_v[b, k * 128 + w4 * 4 + u, :]
                        plsc.store_scatter(tile_v.at[b], [rowv, colv + u], v)
                    return colv + 4

                lax.fori_loop(0, 32, wbody, jnp.zeros((16,), jnp.int32))

        def pair(p, _):
            g0 = p * 2
            load_idx(g0, 0)
            cg0 = fire_gather(0)
            load_idx(g0 + 1, 1)
            cg1 = fire_gather(1)
            cg0.wait()
            retile(0)
            s0 = fire_stores(g0, 0)
            cg1.wait()
            retile(1)
            s1 = fire_stores(g0 + 1, 1)
            for cp in s0 + s1:
                cp.wait()
            return 0

        lax.fori_loop(0, n_chunks // 2, pair, 0)
        if n_chunks % 2:
            g0 = n_chunks - 1
            load_idx(g0, 0)
            fire_gather(0).wait()
            retile(0)
            for cp in fire_stores(g0, 0):
                cp.wait()

    return emb(idx1, lin)


@jax.jit
def kernel(indices, table):
    (B,) = indices.shape
    V, D = table.shape
    Vp = ((V + 127) // 128) * 128

    idx1 = indices.astype(jnp.int32)
    tableT = table.T

    idx2 = _granule_idx(idx1)
    lin2d = _convert(tableT, V, D)
    Vp2 = lin2d.shape[0] * 8
    lin = lin2d.reshape(Vp2, D)
    out3 = _gather(idx2, lin, B, Vp2, D)
    out4d = out3.reshape(2, B // 128, 8, 128)
    return jnp.transpose(out4d, (1, 3, 0, 2)).reshape(B, D)


# speedup vs baseline: 3.2708x; 1.0067x over previous
import functools

import jax
import jax.numpy as jnp
from jax import lax
from jax.experimental import pallas as pl
from jax.experimental.pallas import tpu as pltpu
from jax.experimental.pallas import tpu_sc as plsc

_NC = 2
_NS = 16
_NW = _NC * _NS
_C = 1024
_BN = 131072


def _convert(tableT, V, D):
    bn = _BN
    G = bn // 8
    grid = (V + bn - 1) // bn
    R = G * grid

    def body(x_ref, o_ref):
        x = x_ref[...]
        xx = jnp.concatenate(
            [x[:, G * k:G * (k + 1)] for k in range(8)], axis=0)
        o_ref[...] = xx.T

    return pl.pallas_call(
        body,
        grid=(grid,),
        in_specs=[pl.BlockSpec((D, bn), lambda i: (0, i))],
        out_specs=pl.BlockSpec((G, 128), lambda i: (i, 0)),
        out_shape=jax.ShapeDtypeStruct((R, 128), jnp.float32),
    )(tableT)


def _granule_idx(idx1):
    bi = 102400
    (B,) = idx1.shape

    G = _BN // 8
    sh = G.bit_length() - 1

    def body(x_ref, o_ref):
        v = x_ref[...]
        o_ref[...] = (v & ~(_BN - 1)) | ((v & (G - 1)) << 3) | ((v >> sh) & 7)

    return pl.pallas_call(
        body,
        grid=(B // bi,),
        in_specs=[pl.BlockSpec((bi,), lambda i: (i,))],
        out_specs=pl.BlockSpec((bi,), lambda i: (i,)),
        out_shape=jax.ShapeDtypeStruct((B,), jnp.int32),
    )(idx1)


def _gather(idx1, lin, B, Vp, D):
    b_per_w = B // _NW
    n_chunks = b_per_w // _C
    n_t = _C // 128
    n_tc = B // 128

    mesh = plsc.VectorSubcoreMesh(core_axis_name="c", subcore_axis_name="s")

    @functools.partial(
        pl.kernel,
        mesh=mesh,
        out_type=jax.ShapeDtypeStruct((2, n_tc * 8, 128), jnp.float32),
        scratch_types=[
            pltpu.VMEM((2, _C), jnp.int32),
            pltpu.VMEM((2, _C, D), jnp.float32),
            pltpu.VMEM((2, 2 * n_t * 8, 129), jnp.float32),
            [pltpu.SemaphoreType.DMA] * 2,
            [pltpu.SemaphoreType.DMA] * 2,
        ],
        compiler_params=pltpu.CompilerParams(
            use_tc_tiling_on_sc=False, needs_layout_passes=False),
    )
    def emb(idx_hbm, lin_hbm, out_hbm, idx_v, rows_v, tile_v, gsems, ssems):
        wid = lax.axis_index("s") * _NC + lax.axis_index("c")
        base = wid * b_per_w
        iota = lax.iota(jnp.int32, 16)
        row_f = (iota // 8) * (n_t * 8) + (iota % 8)

        def load_idx(g, b):
            pltpu.sync_copy(idx_hbm.at[pl.ds(base + g * _C, _C)], idx_v.at[b])

        def fire_gather(b):
            return pltpu.async_copy(
                lin_hbm.at[idx_v.at[b]], rows_v.at[b], gsems[b])

        def fire_stores(g, b):
            tc0 = (base + g * _C) // 128
            return [
                pltpu.async_copy(
                    tile_v.at[b, pl.ds(r * (n_t * 8), n_t * 8), pl.ds(0, 128)],
                    out_hbm.at[r, pl.ds(tc0 * 8, n_t * 8)], ssems[b])
                for r in (0, 1)
            ]

        def retile(b):
            for k in range(n_t):
                rowv = row_f + k * 8

                def wbody(w4, colv, k=k, rowv=rowv):
                    for u in range(4):
                        v = rows_v[b, k * 128 + w4 * 4 + u, :]
                        plsc.store_scatter(tile_v.at[b], [rowv, colv + u], v)
                    return colv + 4

                lax.fori_loop(0, 32, wbody, jnp.zeros((16,), jnp.int32))

        def pair(p, _):
            g0 = p * 2
            load_idx(g0, 0)
            cg0 = fire_gather(0)
            load_idx(g0 + 1, 1)
            cg1 = fire_gather(1)
            cg0.wait()
            retile(0)
            s0 = fire_stores(g0, 0)
            cg1.wait()
            retile(1)
            s1 = fire_stores(g0 + 1, 1)
            for cp in s0 + s1:
                cp.wait()
            return 0

        lax.fori_loop(0, n_chunks // 2, pair, 0)
        if n_chunks % 2:
            g0 = n_chunks - 1
            load_idx(g0, 0)
            fire_gather(0).wait()
            retile(0)
            for cp in fire_stores(g0, 0):
                cp.wait()

    return emb(idx1, lin)


@jax.jit
def kernel(indices, table):
    (B,) = indices.shape
    V, D = table.shape
    Vp = ((V + 127) // 128) * 128

    idx1 = indices.astype(jnp.int32)
    tableT = table.T

    idx2 = _granule_idx(idx1)
    lin2d = _convert(tableT, V, D)
    Vp2 = lin2d.shape[0] * 8
    lin = lin2d.reshape(Vp2, D)
    out3 = _gather(idx2, lin, B, Vp2, D)
    out4d = out3.reshape(2, B // 128, 8, 128)
    return jnp.transpose(out4d, (1, 3, 0, 2)).reshape(B, D)
